# baseline (speedup 1.0000x reference)
import functools

import jax
import jax.numpy as jnp
from jax import lax
from jax.experimental import pallas as pl
from jax.experimental.pallas import tpu as pltpu
from jax.experimental.pallas import tpu_sc as plsc

POOLED = 7
H = 50
W = 50
C = 256
NLEV = 4
TSTRIDE = 2560
ZERO_ROW = 2500
NTAB = NLEV * NLEV
TROWS = NTAB * TSTRIDE

NC = 2
NS = 16
NW = NC * NS
RH = 500
CELLS_H = 49 * RH
CELLS_HP = 24576
CPW = CELLS_HP // NW
CHUNK = 32
NCHUNK = CPW // CHUNK


def _roll_max(m, s):
    if s == 0:
        return m
    return jnp.maximum(m, jnp.concatenate([m[s:], m[:s]], axis=0))


def _table_body(f_ref, t_ref):
    t = pl.program_id(0)
    a = f_ref[...]

    def make_branch(ky, kx):
        def br(a):
            m = a
            for j in range(ky):
                m = _roll_max(m, 50 * (1 << j))
            for j in range(kx):
                m = _roll_max(m, 1 << j)
            return m
        return br

    branches = [make_branch(ky, kx) for ky in range(NLEV) for kx in range(NLEV)]
    m = lax.switch(t, branches, a)
    t_ref[0:2500, :] = m
    t_ref[2500:TSTRIDE, :] = jnp.zeros((TSTRIDE - 2500, 256), jnp.float32)


def _build_table(feat2d):
    return pl.pallas_call(
        _table_body,
        grid=(NTAB,),
        in_specs=[pl.BlockSpec((H * W, C), lambda t: (0, 0))],
        out_specs=pl.BlockSpec((TSTRIDE, C), lambda t: (t, 0)),
        out_shape=jax.ShapeDtypeStruct((TROWS, C), jnp.float32),
    )(feat2d)


CW = C // 2


def _pool_body(table_hbm, idx_hbm, out_hbm, idx_v, rows_v, out_v,
               gsem0, gsem1, osem0, osem1):
    wid = lax.axis_index("s") * NC + lax.axis_index("c")
    chunk0 = wid * NCHUNK
    gsems = (gsem0, gsem1)
    osems = (osem0, osem1)

    pltpu.sync_copy(idx_hbm.at[pl.ds(chunk0, NCHUNK)], idx_v)

    def fire_gather(g, b):
        pltpu.async_copy(table_hbm.at[idx_v.at[g]], rows_v.at[b], gsems[b])

    def wait_gather(b):
        pltpu.make_async_copy(table_hbm.at[pl.ds(0, CHUNK * 4)],
                              rows_v.at[b], gsems[b]).wait()

    def fire_out(g, b):
        cb = (chunk0 + g) * CHUNK
        pltpu.async_copy(out_v.at[b], out_hbm.at[pl.ds(cb, CHUNK)], osems[b])

    def wait_out(b):
        pltpu.make_async_copy(out_v.at[b], out_hbm.at[pl.ds(0, CHUNK)],
                              osems[b]).wait()

    fire_gather(0, 0)
    fire_gather(1, 1)

    def group_body(gg, carry):
        for b in (0, 1):
            g = 2 * gg + b
            wait_gather(b)

            @pl.when(gg > 0)
            def _():
                wait_out(b)

            def cell_body(i, c2):
                for c16 in range(C // 16):
                    sl = pl.ds(c16 * 16, 16)
                    v0 = rows_v[b, 4 * i, sl]
                    v1 = rows_v[b, 4 * i + 1, sl]
                    v2 = rows_v[b, 4 * i + 2, sl]
                    v3 = rows_v[b, 4 * i + 3, sl]
                    out_v[b, i, sl] = jnp.maximum(jnp.maximum(v0, v1),
                                                  jnp.maximum(v2, v3))
                return c2

            lax.fori_loop(0, CHUNK, cell_body, 0)
            fire_out(g, b)

            @pl.when(g + 2 < NCHUNK)
            def _():
                fire_gather(g + 2, b)
        return carry

    lax.fori_loop(0, NCHUNK // 2, group_body, 0)
    wait_out(0)
    wait_out(1)


def _pool_sc(table_w, idx):
    mesh = plsc.VectorSubcoreMesh(core_axis_name="c", subcore_axis_name="s")
    f = functools.partial(
        pl.kernel,
        mesh=mesh,
        out_type=jax.ShapeDtypeStruct((CELLS_HP, C), jnp.float32),
        scratch_types=[
            pltpu.VMEM((NCHUNK, CHUNK * 4), jnp.int32),
            pltpu.VMEM((2, CHUNK * 4, C), jnp.float32),
            pltpu.VMEM((2, CHUNK, C), jnp.float32),
            pltpu.SemaphoreType.DMA,
            pltpu.SemaphoreType.DMA,
            pltpu.SemaphoreType.DMA,
            pltpu.SemaphoreType.DMA,
        ],
    )(_pool_body)
    return f(table_w, idx)


def _fc_body(x_ref, w1_ref, w2_ref, wh_ref, b1_ref, b2_ref, bh_ref,
             out_ref, acc_ref):
    j = pl.program_id(0)
    part = lax.dot_general(x_ref[...].astype(jnp.bfloat16), w1_ref[...],
                           (((1,), (1,)), ((), ())),
                           preferred_element_type=jnp.float32)

    @pl.when(j == 0)
    def _():
        acc_ref[...] = part

    @pl.when(j > 0)
    def _():
        acc_ref[...] += part

    @pl.when(j == POOLED * POOLED - 1)
    def _():
        h1 = jnp.maximum(acc_ref[...] + b1_ref[...], 0.0)
        h2 = lax.dot_general(h1, w2_ref[...], (((1,), (1,)), ((), ())),
                             preferred_element_type=jnp.float32)
        h2 = jnp.maximum(h2 + b2_ref[...], 0.0)
        o = lax.dot_general(h2, wh_ref[...], (((1,), (1,)), ((), ())),
                            preferred_element_type=jnp.float32)
        out_ref[...] = o + bh_ref[...]


def _fc_stack(x, w1p, w2, wh, b1, b2, bh):
    R = x.shape[0]
    fc = w2.shape[0]
    return pl.pallas_call(
        _fc_body,
        grid=(POOLED * POOLED,),
        in_specs=[
            pl.BlockSpec((R, C), lambda j: (0, j)),
            pl.BlockSpec((fc, C), lambda j: (0, j)),
            pl.BlockSpec((fc, fc), lambda j: (0, 0)),
            pl.BlockSpec((128, fc), lambda j: (0, 0)),
            pl.BlockSpec((1, fc), lambda j: (0, 0)),
            pl.BlockSpec((1, fc), lambda j: (0, 0)),
            pl.BlockSpec((1, 128), lambda j: (0, 0)),
        ],
        out_specs=pl.BlockSpec((R, 128), lambda j: (0, 0)),
        out_shape=jax.ShapeDtypeStruct((R, 128), jnp.float32),
        scratch_shapes=[pltpu.VMEM((R, fc), jnp.float32)],
    )(x, w1p, w2, wh, b1, b2, bh)


def _cell_indices(proposals, image_shape):
    img_h = image_shape[0].astype(jnp.float32)
    img_w = image_shape[1].astype(jnp.float32)
    scale = jnp.minimum(H / img_h, W / img_w)

    rsw = jnp.round(proposals[:, 0] * scale)
    rsh = jnp.round(proposals[:, 1] * scale)
    rew = jnp.round(proposals[:, 2] * scale)
    reh = jnp.round(proposals[:, 3] * scale)
    roi_w = jnp.maximum(rew - rsw + 1.0, 1.0)
    roi_h = jnp.maximum(reh - rsh + 1.0, 1.0)

    pidx = jnp.arange(POOLED, dtype=jnp.float32)
    hstart = jnp.clip(jnp.floor(pidx[None, :] * roi_h[:, None] / POOLED)
                      + rsh[:, None], 0, H)
    hend = jnp.clip(jnp.ceil((pidx[None, :] + 1.0) * roi_h[:, None] / POOLED)
                    + rsh[:, None], 0, H)
    wstart = jnp.clip(jnp.floor(pidx[None, :] * roi_w[:, None] / POOLED)
                      + rsw[:, None], 0, W)
    wend = jnp.clip(jnp.ceil((pidx[None, :] + 1.0) * roi_w[:, None] / POOLED)
                    + rsw[:, None], 0, W)

    hs = hstart.astype(jnp.int32)
    he = hend.astype(jnp.int32)
    ws = wstart.astype(jnp.int32)
    we = wend.astype(jnp.int32)
    lh = he - hs
    lw = we - ws

    def level(n):
        return jnp.where(n > 8, 3, jnp.where(n > 4, 2, jnp.where(n > 2, 1, 0)))

    kh = level(lh)
    kw = level(lw)
    y0 = hs
    y1 = he - (1 << kh).astype(jnp.int32)
    x0 = ws
    x1 = we - (1 << kw).astype(jnp.int32)

    base = (kh[:, :, None] * NLEV + kw[:, None, :]) * TSTRIDE
    ya = y0[:, :, None] * W
    yb = y1[:, :, None] * W
    xa = x0[:, None, :]
    xb = x1[:, None, :]
    corners = jnp.stack([base + ya + xa, base + ya + xb,
                         base + yb + xa, base + yb + xb], axis=-1)
    empty = (lh[:, :, None] <= 0) | (lw[:, None, :] <= 0)
    corners = jnp.where(empty[..., None], ZERO_ROW, corners)

    flat = corners.reshape(-1, 4).astype(jnp.int32)
    pad = jnp.full(((CELLS_HP - CELLS_H) * 4,), ZERO_ROW, jnp.int32)
    h0 = jnp.concatenate([flat[:CELLS_H].reshape(-1), pad])
    h1 = jnp.concatenate([flat[CELLS_H:].reshape(-1), pad])
    return (h0.reshape(CELLS_HP // CHUNK, CHUNK * 4),
            h1.reshape(CELLS_HP // CHUNK, CHUNK * 4))


def kernel(feat, proposals, image_shape, W1, b1, W2, b2, Wc, bc, Wr, br):
    R = proposals.shape[0]
    fc = W1.shape[0]
    ncls = Wc.shape[0]
    nreg = Wr.shape[0]

    feat2d = jnp.transpose(feat[0], (1, 2, 0)).reshape(H * W, C)
    table = _build_table(feat2d)

    idx0, idx1 = _cell_indices(proposals, image_shape)

    w1p = (W1.astype(jnp.bfloat16)
           .reshape(fc, C, POOLED * POOLED).transpose(0, 2, 1).reshape(fc, -1))

    wh = jnp.zeros((128, fc), jnp.float32)
    wh = lax.dynamic_update_slice(wh, Wc, (0, 0))
    wh = lax.dynamic_update_slice(wh, Wr, (ncls, 0))
    bh = jnp.concatenate([bc, br, jnp.zeros((128 - ncls - nreg,), jnp.float32)])
    b1r, b2r, bhr = b1.reshape(1, fc), b2.reshape(1, fc), bh.reshape(1, 128)

    pooled0 = _pool_sc(table, idx0)
    pooled1 = _pool_sc(table, idx1)
    x0 = pooled0[:CELLS_H].reshape(RH, POOLED * POOLED * C)
    x1 = pooled1[:CELLS_H].reshape(RH, POOLED * POOLED * C)
    out0 = _fc_stack(x0, w1p, W2, wh, b1r, b2r, bhr)
    out1 = _fc_stack(x1, w1p, W2, wh, b1r, b2r, bhr)
    out = jnp.concatenate([out0, out1], axis=0)

    cls_scores = out[:, :ncls]
    box_pred = out[:, ncls:ncls + nreg]
    return cls_scores, box_pred


# speedup vs baseline: 1.1961x; 1.1961x over previous
import functools

import jax
import jax.numpy as jnp
from jax import lax
from jax.experimental import pallas as pl
from jax.experimental.pallas import tpu as pltpu
from jax.experimental.pallas import tpu_sc as plsc

POOLED = 7
H = 50
W = 50
C = 256
NLEV = 4
TSTRIDE = 2560
ZERO_ROW = 2500
NTAB = NLEV * NLEV
TROWS = NTAB * TSTRIDE

NC = 2
NS = 16
NW = NC * NS
CELLS = 49 * 1000
CELLS_PAD = 49152
CPW = CELLS_PAD // NW
CHUNK = 32
NCHUNK = CPW // CHUNK


def _roll_max(m, s):
    if s == 0:
        return m
    return jnp.maximum(m, jnp.concatenate([m[s:], m[:s]], axis=0))


def _table_body(f_ref, t_ref):
    t = pl.program_id(0)
    a = f_ref[...]

    def make_branch(ky, kx):
        def br(a):
            m = a
            for j in range(ky):
                m = _roll_max(m, 50 * (1 << j))
            for j in range(kx):
                m = _roll_max(m, 1 << j)
            return m
        return br

    branches = [make_branch(ky, kx) for ky in range(NLEV) for kx in range(NLEV)]
    m = lax.switch(t, branches, a)
    t_ref[0:2500, :] = m
    t_ref[2500:TSTRIDE, :] = jnp.zeros((TSTRIDE - 2500, 256), jnp.float32)


def _build_table(feat2d):
    return pl.pallas_call(
        _table_body,
        grid=(NTAB,),
        in_specs=[pl.BlockSpec((H * W, C), lambda t: (0, 0))],
        out_specs=pl.BlockSpec((TSTRIDE, C), lambda t: (t, 0)),
        out_shape=jax.ShapeDtypeStruct((TROWS, C), jnp.float32),
    )(feat2d)


CW = C // 2


def _pool_body(table_hbm, idx_hbm, out_hbm, idx_v, rows_v, out_v,
               gsem0, gsem1, osem0, osem1):
    wid = lax.axis_index("s") * NC + lax.axis_index("c")
    chunk0 = wid * NCHUNK
    gsems = (gsem0, gsem1)
    osems = (osem0, osem1)

    pltpu.sync_copy(idx_hbm.at[pl.ds(chunk0, NCHUNK)], idx_v)

    def fire_gather(g, b):
        pltpu.async_copy(table_hbm.at[idx_v.at[g]], rows_v.at[b], gsems[b])

    def wait_gather(b):
        pltpu.make_async_copy(table_hbm.at[pl.ds(0, CHUNK * 4)],
                              rows_v.at[b], gsems[b]).wait()

    def fire_out(g, b):
        cb = (chunk0 + g) * CHUNK
        pltpu.async_copy(out_v.at[b], out_hbm.at[pl.ds(cb, CHUNK)], osems[b])

    def wait_out(b):
        pltpu.make_async_copy(out_v.at[b], out_hbm.at[pl.ds(0, CHUNK)],
                              osems[b]).wait()

    fire_gather(0, 0)
    fire_gather(1, 1)

    def group_body(gg, carry):
        for b in (0, 1):
            g = 2 * gg + b
            wait_gather(b)

            @pl.when(gg > 0)
            def _():
                wait_out(b)

            def cell_body(i, c2):
                for c16 in range(C // 16):
                    sl = pl.ds(c16 * 16, 16)
                    v0 = rows_v[b, 4 * i, sl]
                    v1 = rows_v[b, 4 * i + 1, sl]
                    v2 = rows_v[b, 4 * i + 2, sl]
                    v3 = rows_v[b, 4 * i + 3, sl]
                    out_v[b, i, sl] = jnp.maximum(jnp.maximum(v0, v1),
                                                  jnp.maximum(v2, v3))
                return c2

            lax.fori_loop(0, CHUNK, cell_body, 0)
            fire_out(g, b)

            @pl.when(g + 2 < NCHUNK)
            def _():
                fire_gather(g + 2, b)
        return carry

    lax.fori_loop(0, NCHUNK // 2, group_body, 0)
    wait_out(0)
    wait_out(1)


def _pool_sc(table_w, idx):
    mesh = plsc.VectorSubcoreMesh(core_axis_name="c", subcore_axis_name="s")
    f = functools.partial(
        pl.kernel,
        mesh=mesh,
        out_type=jax.ShapeDtypeStruct((CELLS_PAD, C), jnp.float32),
        scratch_types=[
            pltpu.VMEM((NCHUNK, CHUNK * 4), jnp.int32),
            pltpu.VMEM((2, CHUNK * 4, C), jnp.float32),
            pltpu.VMEM((2, CHUNK, C), jnp.float32),
            pltpu.SemaphoreType.DMA,
            pltpu.SemaphoreType.DMA,
            pltpu.SemaphoreType.DMA,
            pltpu.SemaphoreType.DMA,
        ],
    )(_pool_body)
    return f(table_w, idx.reshape(CELLS_PAD // CHUNK, CHUNK * 4))


def _fc_body(x_ref, w1_ref, w2_ref, wh_ref, b1_ref, b2_ref, bh_ref,
             out_ref, acc_ref):
    j = pl.program_id(0)
    part = lax.dot_general(x_ref[...].astype(jnp.bfloat16), w1_ref[...],
                           (((1,), (1,)), ((), ())),
                           preferred_element_type=jnp.float32)

    @pl.when(j == 0)
    def _():
        acc_ref[...] = part

    @pl.when(j > 0)
    def _():
        acc_ref[...] += part

    @pl.when(j == POOLED * POOLED - 1)
    def _():
        h1 = jnp.maximum(acc_ref[...] + b1_ref[...], 0.0)
        h2 = lax.dot_general(h1, w2_ref[...], (((1,), (1,)), ((), ())),
                             preferred_element_type=jnp.float32)
        h2 = jnp.maximum(h2 + b2_ref[...], 0.0)
        o = lax.dot_general(h2, wh_ref[...], (((1,), (1,)), ((), ())),
                            preferred_element_type=jnp.float32)
        out_ref[...] = o + bh_ref[...]


def _fc_stack(x, w1p, w2, wh, b1, b2, bh, R):
    fc = w2.shape[0]
    return pl.pallas_call(
        _fc_body,
        grid=(POOLED * POOLED,),
        in_specs=[
            pl.BlockSpec((R, C), lambda j: (j, 0)),
            pl.BlockSpec((fc, C), lambda j: (0, j)),
            pl.BlockSpec((fc, fc), lambda j: (0, 0)),
            pl.BlockSpec((128, fc), lambda j: (0, 0)),
            pl.BlockSpec((1, fc), lambda j: (0, 0)),
            pl.BlockSpec((1, fc), lambda j: (0, 0)),
            pl.BlockSpec((1, 128), lambda j: (0, 0)),
        ],
        out_specs=pl.BlockSpec((R, 128), lambda j: (0, 0)),
        out_shape=jax.ShapeDtypeStruct((R, 128), jnp.float32),
        scratch_shapes=[pltpu.VMEM((R, fc), jnp.float32)],
    )(x, w1p, w2, wh, b1, b2, bh)


def _cell_indices(proposals, image_shape):
    img_h = image_shape[0].astype(jnp.float32)
    img_w = image_shape[1].astype(jnp.float32)
    scale = jnp.minimum(H / img_h, W / img_w)

    rsw = jnp.round(proposals[:, 0] * scale)
    rsh = jnp.round(proposals[:, 1] * scale)
    rew = jnp.round(proposals[:, 2] * scale)
    reh = jnp.round(proposals[:, 3] * scale)
    roi_w = jnp.maximum(rew - rsw + 1.0, 1.0)
    roi_h = jnp.maximum(reh - rsh + 1.0, 1.0)

    pidx = jnp.arange(POOLED, dtype=jnp.float32)
    hstart = jnp.clip(jnp.floor(pidx[None, :] * roi_h[:, None] / POOLED)
                      + rsh[:, None], 0, H)
    hend = jnp.clip(jnp.ceil((pidx[None, :] + 1.0) * roi_h[:, None] / POOLED)
                    + rsh[:, None], 0, H)
    wstart = jnp.clip(jnp.floor(pidx[None, :] * roi_w[:, None] / POOLED)
                      + rsw[:, None], 0, W)
    wend = jnp.clip(jnp.ceil((pidx[None, :] + 1.0) * roi_w[:, None] / POOLED)
                    + rsw[:, None], 0, W)

    hs = hstart.astype(jnp.int32)
    he = hend.astype(jnp.int32)
    ws = wstart.astype(jnp.int32)
    we = wend.astype(jnp.int32)
    lh = he - hs
    lw = we - ws

    def level(n):
        return jnp.where(n > 8, 3, jnp.where(n > 4, 2, jnp.where(n > 2, 1, 0)))

    kh = level(lh)
    kw = level(lw)
    y0 = hs
    y1 = he - (1 << kh).astype(jnp.int32)
    x0 = ws
    x1 = we - (1 << kw).astype(jnp.int32)

    base = (kh[:, :, None] * NLEV + kw[:, None, :]) * TSTRIDE
    ya = y0[:, :, None] * W
    yb = y1[:, :, None] * W
    xa = x0[:, None, :]
    xb = x1[:, None, :]
    corners = jnp.stack([base + ya + xa, base + ya + xb,
                         base + yb + xa, base + yb + xb], axis=-1)
    empty = (lh[:, :, None] <= 0) | (lw[:, None, :] <= 0)
    corners = jnp.where(empty[..., None], ZERO_ROW, corners)

    flat = jnp.transpose(corners, (1, 2, 0, 3)).reshape(-1)
    pad = jnp.full(((CELLS_PAD - CELLS) * 4,), ZERO_ROW, jnp.int32)
    return jnp.concatenate([flat.astype(jnp.int32), pad])


def kernel(feat, proposals, image_shape, W1, b1, W2, b2, Wc, bc, Wr, br):
    R = proposals.shape[0]
    fc = W1.shape[0]
    ncls = Wc.shape[0]
    nreg = Wr.shape[0]

    feat2d = jnp.transpose(feat[0], (1, 2, 0)).reshape(H * W, C)
    table = _build_table(feat2d)

    idx = _cell_indices(proposals, image_shape)
    pooled = _pool_sc(table, idx)

    w1p = (W1.astype(jnp.bfloat16)
           .reshape(fc, C, POOLED * POOLED).transpose(0, 2, 1).reshape(fc, -1))

    wh = jnp.zeros((128, fc), jnp.float32)
    wh = lax.dynamic_update_slice(wh, Wc, (0, 0))
    wh = lax.dynamic_update_slice(wh, Wr, (ncls, 0))
    bh = jnp.concatenate([bc, br, jnp.zeros((128 - ncls - nreg,), jnp.float32)])

    out = _fc_stack(pooled, w1p, W2, wh,
                    b1.reshape(1, fc), b2.reshape(1, fc), bh.reshape(1, 128),
                    R)
    cls_scores = out[:, :ncls]
    box_pred = out[:, ncls:ncls + nreg]
    return cls_scores, box_pred


# speedup vs baseline: 1.1963x; 1.0002x over previous
import functools

import jax
import jax.numpy as jnp
from jax import lax
from jax.experimental import pallas as pl
from jax.experimental.pallas import tpu as pltpu
from jax.experimental.pallas import tpu_sc as plsc

POOLED = 7
H = 50
W = 50
C = 256
NLEV = 4
TSTRIDE = 2560
ZERO_ROW = 2500
NTAB = NLEV * NLEV
TROWS = NTAB * TSTRIDE

NC = 2
NS = 16
NW = NC * NS
CELLS = 49 * 1000
CELLS_PAD = 49152
CPW = CELLS_PAD // NW
CHUNK = 32
NCHUNK = CPW // CHUNK


def _roll_max(m, s):
    if s == 0:
        return m
    return jnp.maximum(m, jnp.concatenate([m[s:], m[:s]], axis=0))


def _table_body(f_ref, t_ref):
    t = pl.program_id(0)
    a = f_ref[...]

    def make_branch(ky, kx):
        def br(a):
            m = a
            for j in range(ky):
                m = _roll_max(m, 50 * (1 << j))
            for j in range(kx):
                m = _roll_max(m, 1 << j)
            return m
        return br

    branches = [make_branch(ky, kx) for ky in range(NLEV) for kx in range(NLEV)]
    m = lax.switch(t, branches, a)
    t_ref[0:2500, :] = m
    t_ref[2500:TSTRIDE, :] = jnp.zeros((TSTRIDE - 2500, 256), jnp.float32)


def _build_table(feat2d):
    return pl.pallas_call(
        _table_body,
        grid=(NTAB,),
        in_specs=[pl.BlockSpec((H * W, C), lambda t: (0, 0))],
        out_specs=pl.BlockSpec((TSTRIDE, C), lambda t: (t, 0)),
        out_shape=jax.ShapeDtypeStruct((TROWS, C), jnp.float32),
    )(feat2d)


CW = C // 2


def _pool_body(table_hbm, idx_hbm, out_hbm, idx_v, rows_v, out_v,
               gsem0, gsem1, osem0, osem1):
    wid = lax.axis_index("s") * NC + lax.axis_index("c")
    chunk0 = wid * NCHUNK
    gsems = (gsem0, gsem1)
    osems = (osem0, osem1)

    pltpu.sync_copy(idx_hbm.at[pl.ds(chunk0, NCHUNK)], idx_v)

    def fire_gather(g, b):
        pltpu.async_copy(table_hbm.at[idx_v.at[g]], rows_v.at[b], gsems[b])

    def wait_gather(b):
        pltpu.make_async_copy(table_hbm.at[pl.ds(0, CHUNK * 4)],
                              rows_v.at[b], gsems[b]).wait()

    def fire_out(g, b):
        cb = (chunk0 + g) * CHUNK
        pltpu.async_copy(out_v.at[b], out_hbm.at[pl.ds(cb, CHUNK)], osems[b])

    def wait_out(b):
        pltpu.make_async_copy(out_v.at[b], out_hbm.at[pl.ds(0, CHUNK)],
                              osems[b]).wait()

    fire_gather(0, 0)
    fire_gather(1, 1)

    def group_body(gg, carry):
        for b in (0, 1):
            g = 2 * gg + b
            wait_gather(b)

            @pl.when(gg > 0)
            def _():
                wait_out(b)

            def cell_body(i, c2):
                for c16 in range(C // 16):
                    sl = pl.ds(c16 * 16, 16)
                    v0 = rows_v[b, 4 * i, sl]
                    v1 = rows_v[b, 4 * i + 1, sl]
                    v2 = rows_v[b, 4 * i + 2, sl]
                    v3 = rows_v[b, 4 * i + 3, sl]
                    out_v[b, i, sl] = jnp.maximum(jnp.maximum(v0, v1),
                                                  jnp.maximum(v2, v3))
                return c2

            lax.fori_loop(0, CHUNK, cell_body, 0)
            fire_out(g, b)

            @pl.when(g + 2 < NCHUNK)
            def _():
                fire_gather(g + 2, b)
        return carry

    lax.fori_loop(0, NCHUNK // 2, group_body, 0)
    wait_out(0)
    wait_out(1)


def _pool_sc(table_w, idx):
    mesh = plsc.VectorSubcoreMesh(core_axis_name="c", subcore_axis_name="s")
    f = functools.partial(
        pl.kernel,
        mesh=mesh,
        compiler_params=pltpu.CompilerParams(use_tc_tiling_on_sc=True),
        out_type=jax.ShapeDtypeStruct((CELLS_PAD, C), jnp.float32),
        scratch_types=[
            pltpu.VMEM((NCHUNK, CHUNK * 4), jnp.int32),
            pltpu.VMEM((2, CHUNK * 4, C), jnp.float32),
            pltpu.VMEM((2, CHUNK, C), jnp.float32),
            pltpu.SemaphoreType.DMA,
            pltpu.SemaphoreType.DMA,
            pltpu.SemaphoreType.DMA,
            pltpu.SemaphoreType.DMA,
        ],
    )(_pool_body)
    return f(table_w, idx.reshape(CELLS_PAD // CHUNK, CHUNK * 4))


def _fc_body(x_ref, w1_ref, w2_ref, wh_ref, b1_ref, b2_ref, bh_ref,
             out_ref, acc_ref):
    j = pl.program_id(0)
    part = lax.dot_general(x_ref[...].astype(jnp.bfloat16), w1_ref[...],
                           (((1,), (1,)), ((), ())),
                           preferred_element_type=jnp.float32)

    @pl.when(j == 0)
    def _():
        acc_ref[...] = part

    @pl.when(j > 0)
    def _():
        acc_ref[...] += part

    @pl.when(j == POOLED * POOLED - 1)
    def _():
        h1 = jnp.maximum(acc_ref[...] + b1_ref[...], 0.0)
        h2 = lax.dot_general(h1, w2_ref[...], (((1,), (1,)), ((), ())),
                             preferred_element_type=jnp.float32)
        h2 = jnp.maximum(h2 + b2_ref[...], 0.0)
        o = lax.dot_general(h2, wh_ref[...], (((1,), (1,)), ((), ())),
                            preferred_element_type=jnp.float32)
        out_ref[...] = o + bh_ref[...]


def _fc_stack(x, w1p, w2, wh, b1, b2, bh, R):
    fc = w2.shape[0]
    return pl.pallas_call(
        _fc_body,
        grid=(POOLED * POOLED,),
        in_specs=[
            pl.BlockSpec((R, C), lambda j: (j, 0)),
            pl.BlockSpec((fc, C), lambda j: (0, j)),
            pl.BlockSpec((fc, fc), lambda j: (0, 0)),
            pl.BlockSpec((128, fc), lambda j: (0, 0)),
            pl.BlockSpec((1, fc), lambda j: (0, 0)),
            pl.BlockSpec((1, fc), lambda j: (0, 0)),
            pl.BlockSpec((1, 128), lambda j: (0, 0)),
        ],
        out_specs=pl.BlockSpec((R, 128), lambda j: (0, 0)),
        out_shape=jax.ShapeDtypeStruct((R, 128), jnp.float32),
        scratch_shapes=[pltpu.VMEM((R, fc), jnp.float32)],
    )(x, w1p, w2, wh, b1, b2, bh)


def _cell_indices(proposals, image_shape):
    img_h = image_shape[0].astype(jnp.float32)
    img_w = image_shape[1].astype(jnp.float32)
    scale = jnp.minimum(H / img_h, W / img_w)

    rsw = jnp.round(proposals[:, 0] * scale)
    rsh = jnp.round(proposals[:, 1] * scale)
    rew = jnp.round(proposals[:, 2] * scale)
    reh = jnp.round(proposals[:, 3] * scale)
    roi_w = jnp.maximum(rew - rsw + 1.0, 1.0)
    roi_h = jnp.maximum(reh - rsh + 1.0, 1.0)

    pidx = jnp.arange(POOLED, dtype=jnp.float32)
    hstart = jnp.clip(jnp.floor(pidx[None, :] * roi_h[:, None] / POOLED)
                      + rsh[:, None], 0, H)
    hend = jnp.clip(jnp.ceil((pidx[None, :] + 1.0) * roi_h[:, None] / POOLED)
                    + rsh[:, None], 0, H)
    wstart = jnp.clip(jnp.floor(pidx[None, :] * roi_w[:, None] / POOLED)
                      + rsw[:, None], 0, W)
    wend = jnp.clip(jnp.ceil((pidx[None, :] + 1.0) * roi_w[:, None] / POOLED)
                    + rsw[:, None], 0, W)

    hs = hstart.astype(jnp.int32)
    he = hend.astype(jnp.int32)
    ws = wstart.astype(jnp.int32)
    we = wend.astype(jnp.int32)
    lh = he - hs
    lw = we - ws

    def level(n):
        return jnp.where(n > 8, 3, jnp.where(n > 4, 2, jnp.where(n > 2, 1, 0)))

    kh = level(lh)
    kw = level(lw)
    y0 = hs
    y1 = he - (1 << kh).astype(jnp.int32)
    x0 = ws
    x1 = we - (1 << kw).astype(jnp.int32)

    base = (kh[:, :, None] * NLEV + kw[:, None, :]) * TSTRIDE
    ya = y0[:, :, None] * W
    yb = y1[:, :, None] * W
    xa = x0[:, None, :]
    xb = x1[:, None, :]
    corners = jnp.stack([base + ya + xa, base + ya + xb,
                         base + yb + xa, base + yb + xb], axis=-1)
    empty = (lh[:, :, None] <= 0) | (lw[:, None, :] <= 0)
    corners = jnp.where(empty[..., None], ZERO_ROW, corners)

    flat = jnp.transpose(corners, (1, 2, 0, 3)).reshape(-1)
    pad = jnp.full(((CELLS_PAD - CELLS) * 4,), ZERO_ROW, jnp.int32)
    return jnp.concatenate([flat.astype(jnp.int32), pad])


def kernel(feat, proposals, image_shape, W1, b1, W2, b2, Wc, bc, Wr, br):
    R = proposals.shape[0]
    fc = W1.shape[0]
    ncls = Wc.shape[0]
    nreg = Wr.shape[0]

    feat2d = jnp.transpose(feat[0], (1, 2, 0)).reshape(H * W, C)
    table = _build_table(feat2d)

    idx = _cell_indices(proposals, image_shape)
    pooled = _pool_sc(table, idx)

    w1p = (W1.astype(jnp.bfloat16)
           .reshape(fc, C, POOLED * POOLED).transpose(0, 2, 1).reshape(fc, -1))

    wh = jnp.zeros((128, fc), jnp.float32)
    wh = lax.dynamic_update_slice(wh, Wc, (0, 0))
    wh = lax.dynamic_update_slice(wh, Wr, (ncls, 0))
    bh = jnp.concatenate([bc, br, jnp.zeros((128 - ncls - nreg,), jnp.float32)])

    out = _fc_stack(pooled, w1p, W2, wh,
                    b1.reshape(1, fc), b2.reshape(1, fc), bh.reshape(1, 128),
                    R)
    cls_scores = out[:, :ncls]
    box_pred = out[:, ncls:ncls + nreg]
    return cls_scores, box_pred


# speedup vs baseline: 1.1968x; 1.0004x over previous
import functools

import jax
import jax.numpy as jnp
from jax import lax
from jax.experimental import pallas as pl
from jax.experimental.pallas import tpu as pltpu
from jax.experimental.pallas import tpu_sc as plsc

POOLED = 7
H = 50
W = 50
C = 256
NLEV = 4
TSTRIDE = 2560
ZERO_ROW = 2500
NTAB = NLEV * NLEV
TROWS = NTAB * TSTRIDE

NC = 2
NS = 16
NW = NC * NS
CELLS = 49 * 1000
CELLS_PAD = 49152
CPW = CELLS_PAD // NW
CHUNK = 32
NCHUNK = CPW // CHUNK


def _roll_max(m, s):
    if s == 0:
        return m
    return jnp.maximum(m, jnp.concatenate([m[s:], m[:s]], axis=0))


def _table_body(f_ref, t_ref):
    t = pl.program_id(0)
    a = f_ref[...]

    def make_branch(ky, kx):
        def br(a):
            m = a
            for j in range(ky):
                m = _roll_max(m, 50 * (1 << j))
            for j in range(kx):
                m = _roll_max(m, 1 << j)
            return m
        return br

    branches = [make_branch(ky, kx) for ky in range(NLEV) for kx in range(NLEV)]
    m = lax.switch(t, branches, a)
    t_ref[0:2500, :] = m
    t_ref[2500:TSTRIDE, :] = jnp.zeros((TSTRIDE - 2500, 256), jnp.float32)


def _build_table(feat2d):
    return pl.pallas_call(
        _table_body,
        grid=(NTAB,),
        in_specs=[pl.BlockSpec((H * W, C), lambda t: (0, 0))],
        out_specs=pl.BlockSpec((TSTRIDE, C), lambda t: (t, 0)),
        out_shape=jax.ShapeDtypeStruct((TROWS, C), jnp.float32),
    )(feat2d)


CW = C // 2


def _pool_body(table_hbm, idx_hbm, out_hbm, idx_v, rows_v, out_v,
               gsem0, gsem1, osem0, osem1):
    wid = lax.axis_index("s") * NC + lax.axis_index("c")
    chunk0 = wid * NCHUNK
    gsems = (gsem0, gsem1)
    osems = (osem0, osem1)

    pltpu.sync_copy(idx_hbm.at[pl.ds(chunk0, NCHUNK)], idx_v)

    def fire_gather(g, b):
        pltpu.async_copy(table_hbm.at[idx_v.at[g]], rows_v.at[b], gsems[b])

    def wait_gather(b):
        pltpu.make_async_copy(table_hbm.at[pl.ds(0, CHUNK * 4)],
                              rows_v.at[b], gsems[b]).wait()

    def fire_out(g, b):
        cb = (chunk0 + g) * CHUNK
        pltpu.async_copy(out_v.at[b], out_hbm.at[pl.ds(cb, CHUNK)], osems[b])

    def wait_out(b):
        pltpu.make_async_copy(out_v.at[b], out_hbm.at[pl.ds(0, CHUNK)],
                              osems[b]).wait()

    fire_gather(0, 0)
    fire_gather(1, 1)

    def group_body(gg, carry):
        for b in (0, 1):
            g = 2 * gg + b
            wait_gather(b)

            @pl.when(gg > 0)
            def _():
                wait_out(b)

            def cell_body(i2, c2):
                for u in range(2):
                    i = 2 * i2 + u
                    for c16 in range(C // 16):
                        sl = pl.ds(c16 * 16, 16)
                        v0 = rows_v[b, 4 * i, sl]
                        v1 = rows_v[b, 4 * i + 1, sl]
                        v2 = rows_v[b, 4 * i + 2, sl]
                        v3 = rows_v[b, 4 * i + 3, sl]
                        out_v[b, i, sl] = jnp.maximum(jnp.maximum(v0, v1),
                                                      jnp.maximum(v2, v3))
                return c2

            lax.fori_loop(0, CHUNK // 2, cell_body, 0)
            fire_out(g, b)

            @pl.when(g + 2 < NCHUNK)
            def _():
                fire_gather(g + 2, b)
        return carry

    lax.fori_loop(0, NCHUNK // 2, group_body, 0)
    wait_out(0)
    wait_out(1)


def _pool_sc(table_w, idx):
    mesh = plsc.VectorSubcoreMesh(core_axis_name="c", subcore_axis_name="s")
    f = functools.partial(
        pl.kernel,
        mesh=mesh,
        out_type=jax.ShapeDtypeStruct((CELLS_PAD, C), jnp.float32),
        scratch_types=[
            pltpu.VMEM((NCHUNK, CHUNK * 4), jnp.int32),
            pltpu.VMEM((2, CHUNK * 4, C), jnp.float32),
            pltpu.VMEM((2, CHUNK, C), jnp.float32),
            pltpu.SemaphoreType.DMA,
            pltpu.SemaphoreType.DMA,
            pltpu.SemaphoreType.DMA,
            pltpu.SemaphoreType.DMA,
        ],
    )(_pool_body)
    return f(table_w, idx.reshape(CELLS_PAD // CHUNK, CHUNK * 4))


def _fc_body(x_ref, w1_ref, w2_ref, wh_ref, b1_ref, b2_ref, bh_ref,
             out_ref, acc_ref):
    j = pl.program_id(0)
    part = lax.dot_general(x_ref[...].astype(jnp.bfloat16), w1_ref[...],
                           (((1,), (1,)), ((), ())),
                           preferred_element_type=jnp.float32)

    @pl.when(j == 0)
    def _():
        acc_ref[...] = part

    @pl.when(j > 0)
    def _():
        acc_ref[...] += part

    @pl.when(j == POOLED * POOLED - 1)
    def _():
        h1 = jnp.maximum(acc_ref[...] + b1_ref[...], 0.0)
        h2 = lax.dot_general(h1, w2_ref[...], (((1,), (1,)), ((), ())),
                             preferred_element_type=jnp.float32)
        h2 = jnp.maximum(h2 + b2_ref[...], 0.0)
        o = lax.dot_general(h2, wh_ref[...], (((1,), (1,)), ((), ())),
                            preferred_element_type=jnp.float32)
        out_ref[...] = o + bh_ref[...]


def _fc_stack(x, w1p, w2, wh, b1, b2, bh, R):
    fc = w2.shape[0]
    return pl.pallas_call(
        _fc_body,
        grid=(POOLED * POOLED,),
        in_specs=[
            pl.BlockSpec((R, C), lambda j: (j, 0)),
            pl.BlockSpec((fc, C), lambda j: (0, j)),
            pl.BlockSpec((fc, fc), lambda j: (0, 0)),
            pl.BlockSpec((128, fc), lambda j: (0, 0)),
            pl.BlockSpec((1, fc), lambda j: (0, 0)),
            pl.BlockSpec((1, fc), lambda j: (0, 0)),
            pl.BlockSpec((1, 128), lambda j: (0, 0)),
        ],
        out_specs=pl.BlockSpec((R, 128), lambda j: (0, 0)),
        out_shape=jax.ShapeDtypeStruct((R, 128), jnp.float32),
        scratch_shapes=[pltpu.VMEM((R, fc), jnp.float32)],
    )(x, w1p, w2, wh, b1, b2, bh)


def _cell_indices(proposals, image_shape):
    img_h = image_shape[0].astype(jnp.float32)
    img_w = image_shape[1].astype(jnp.float32)
    scale = jnp.minimum(H / img_h, W / img_w)

    rsw = jnp.round(proposals[:, 0] * scale)
    rsh = jnp.round(proposals[:, 1] * scale)
    rew = jnp.round(proposals[:, 2] * scale)
    reh = jnp.round(proposals[:, 3] * scale)
    roi_w = jnp.maximum(rew - rsw + 1.0, 1.0)
    roi_h = jnp.maximum(reh - rsh + 1.0, 1.0)

    pidx = jnp.arange(POOLED, dtype=jnp.float32)
    hstart = jnp.clip(jnp.floor(pidx[None, :] * roi_h[:, None] / POOLED)
                      + rsh[:, None], 0, H)
    hend = jnp.clip(jnp.ceil((pidx[None, :] + 1.0) * roi_h[:, None] / POOLED)
                    + rsh[:, None], 0, H)
    wstart = jnp.clip(jnp.floor(pidx[None, :] * roi_w[:, None] / POOLED)
                      + rsw[:, None], 0, W)
    wend = jnp.clip(jnp.ceil((pidx[None, :] + 1.0) * roi_w[:, None] / POOLED)
                    + rsw[:, None], 0, W)

    hs = hstart.astype(jnp.int32)
    he = hend.astype(jnp.int32)
    ws = wstart.astype(jnp.int32)
    we = wend.astype(jnp.int32)
    lh = he - hs
    lw = we - ws

    def level(n):
        return jnp.where(n > 8, 3, jnp.where(n > 4, 2, jnp.where(n > 2, 1, 0)))

    kh = level(lh)
    kw = level(lw)
    y0 = hs
    y1 = he - (1 << kh).astype(jnp.int32)
    x0 = ws
    x1 = we - (1 << kw).astype(jnp.int32)

    base = (kh[:, :, None] * NLEV + kw[:, None, :]) * TSTRIDE
    ya = y0[:, :, None] * W
    yb = y1[:, :, None] * W
    xa = x0[:, None, :]
    xb = x1[:, None, :]
    corners = jnp.stack([base + ya + xa, base + ya + xb,
                         base + yb + xa, base + yb + xb], axis=-1)
    empty = (lh[:, :, None] <= 0) | (lw[:, None, :] <= 0)
    corners = jnp.where(empty[..., None], ZERO_ROW, corners)

    flat = jnp.transpose(corners, (1, 2, 0, 3)).reshape(-1)
    pad = jnp.full(((CELLS_PAD - CELLS) * 4,), ZERO_ROW, jnp.int32)
    return jnp.concatenate([flat.astype(jnp.int32), pad])


def kernel(feat, proposals, image_shape, W1, b1, W2, b2, Wc, bc, Wr, br):
    R = proposals.shape[0]
    fc = W1.shape[0]
    ncls = Wc.shape[0]
    nreg = Wr.shape[0]

    feat2d = jnp.transpose(feat[0], (1, 2, 0)).reshape(H * W, C)
    table = _build_table(feat2d)

    idx = _cell_indices(proposals, image_shape)
    pooled = _pool_sc(table, idx)

    w1p = (W1.astype(jnp.bfloat16)
           .reshape(fc, C, POOLED * POOLED).transpose(0, 2, 1).reshape(fc, -1))

    wh = jnp.zeros((128, fc), jnp.float32)
    wh = lax.dynamic_update_slice(wh, Wc, (0, 0))
    wh = lax.dynamic_update_slice(wh, Wr, (ncls, 0))
    bh = jnp.concatenate([bc, br, jnp.zeros((128 - ncls - nreg,), jnp.float32)])

    out = _fc_stack(pooled, w1p, W2, wh,
                    b1.reshape(1, fc), b2.reshape(1, fc), bh.reshape(1, 128),
                    R)
    cls_scores = out[:, :ncls]
    box_pred = out[:, ncls:ncls + nreg]
    return cls_scores, box_pred
